# baseline (speedup 1.0000x reference)
import functools

import jax
import jax.numpy as jnp
from jax import lax
from jax.experimental import pallas as pl
from jax.experimental.pallas import tpu as pltpu
from jax.experimental.pallas import tpu_sc as plsc

VOCAB = 100000
EMB = 64
BATCH = 1024

_NUM_CORES = 2
_NUM_SUBCORES = 16
_NUM_WORKERS = _NUM_CORES * _NUM_SUBCORES
_ROWS_PER_W = EMB // _NUM_WORKERS
_LANES = 16

_VTILE = 4864


def _sc_gather_cols(target, table_t):
    mesh = plsc.VectorSubcoreMesh(core_axis_name="c", subcore_axis_name="s")

    @functools.partial(
        pl.kernel,
        mesh=mesh,
        compiler_params=pltpu.CompilerParams(needs_layout_passes=False),
        out_type=jax.ShapeDtypeStruct((EMB, BATCH), jnp.float32),
        scratch_types=[
            pltpu.VMEM((BATCH,), jnp.int32),
            pltpu.VMEM((VOCAB,), jnp.float32),
            pltpu.VMEM((BATCH,), jnp.float32),
            pltpu.SemaphoreType.DMA,
        ],
    )
    def gather_kernel(idx_hbm, table_hbm, out_hbm, idx_v, row_v, out_v, sem):
        wid = lax.axis_index("s") * _NUM_CORES + lax.axis_index("c")
        pltpu.sync_copy(idx_hbm, idx_v)
        for r in range(_ROWS_PER_W):
            e = wid * _ROWS_PER_W + r
            pltpu.sync_copy(table_hbm.at[e], row_v)
            for k in range(BATCH // _LANES):
                sl = pl.ds(k * _LANES, _LANES)
                out_v[sl] = plsc.load_gather(row_v, [idx_v[sl]])
            pltpu.sync_copy(out_v, out_hbm.at[e])

    return gather_kernel(target, table_t)


def _proj_body(x_ref, wt_ref, b_ref, out_ref):
    acc = lax.dot_general(
        wt_ref[...],
        x_ref[...],
        (((0,), (0,)), ((), ())),
        preferred_element_type=jnp.float32,
    )
    out_ref[...] = acc + b_ref[...].T


def _tc_project_t(x, Wt, b2d):
    n_vtiles = pl.cdiv(VOCAB, _VTILE)
    return pl.pallas_call(
        _proj_body,
        grid=(n_vtiles,),
        in_specs=[
            pl.BlockSpec((EMB, BATCH), lambda v: (0, 0)),
            pl.BlockSpec((EMB, _VTILE), lambda v: (0, v)),
            pl.BlockSpec((1, _VTILE), lambda v: (0, v)),
        ],
        out_specs=pl.BlockSpec((_VTILE, BATCH), lambda v: (v, 0)),
        out_shape=jax.ShapeDtypeStruct((VOCAB, BATCH), jnp.float32),
    )(x, Wt, b2d)


def kernel(target, emb_table, W, b):
    target = target.astype(jnp.int32)
    x = _sc_gather_cols(target, emb_table.T)
    out_t = _tc_project_t(x, W.T, b.reshape(1, VOCAB))
    return out_t.T


# speedup vs baseline: 1.0005x; 1.0005x over previous
import functools

import jax
import jax.numpy as jnp
from jax import lax
from jax.experimental import pallas as pl
from jax.experimental.pallas import tpu as pltpu
from jax.experimental.pallas import tpu_sc as plsc

VOCAB = 100000
EMB = 64
BATCH = 1024

_NUM_CORES = 2
_NUM_SUBCORES = 16
_NUM_WORKERS = _NUM_CORES * _NUM_SUBCORES
_ROWS_PER_W = EMB // _NUM_WORKERS
_LANES = 16

_VTILE = 4096


def _sc_gather_cols(target, table_t):
    mesh = plsc.VectorSubcoreMesh(core_axis_name="c", subcore_axis_name="s")

    @functools.partial(
        pl.kernel,
        mesh=mesh,
        compiler_params=pltpu.CompilerParams(needs_layout_passes=False),
        out_type=jax.ShapeDtypeStruct((EMB, BATCH), jnp.float32),
        scratch_types=[
            pltpu.VMEM((BATCH,), jnp.int32),
            pltpu.VMEM((VOCAB,), jnp.float32),
            pltpu.VMEM((BATCH,), jnp.float32),
            pltpu.SemaphoreType.DMA,
        ],
    )
    def gather_kernel(idx_hbm, table_hbm, out_hbm, idx_v, row_v, out_v, sem):
        wid = lax.axis_index("s") * _NUM_CORES + lax.axis_index("c")
        pltpu.sync_copy(idx_hbm, idx_v)
        for r in range(_ROWS_PER_W):
            e = wid * _ROWS_PER_W + r
            pltpu.sync_copy(table_hbm.at[e], row_v)
            for k in range(BATCH // _LANES):
                sl = pl.ds(k * _LANES, _LANES)
                out_v[sl] = plsc.load_gather(row_v, [idx_v[sl]])
            pltpu.sync_copy(out_v, out_hbm.at[e])

    return gather_kernel(target, table_t)


def _proj_body(x_ref, wt_ref, b_ref, out_ref):
    acc = lax.dot_general(
        wt_ref[...],
        x_ref[...],
        (((0,), (0,)), ((), ())),
        preferred_element_type=jnp.float32,
    )
    out_ref[...] = acc + b_ref[...].T


def _tc_project_t(x, Wt, b2d):
    n_vtiles = pl.cdiv(VOCAB, _VTILE)
    return pl.pallas_call(
        _proj_body,
        grid=(n_vtiles,),
        in_specs=[
            pl.BlockSpec((EMB, BATCH), lambda v: (0, 0)),
            pl.BlockSpec((EMB, _VTILE), lambda v: (0, v)),
            pl.BlockSpec((1, _VTILE), lambda v: (0, v)),
        ],
        out_specs=pl.BlockSpec((_VTILE, BATCH), lambda v: (v, 0)),
        out_shape=jax.ShapeDtypeStruct((VOCAB, BATCH), jnp.float32),
    )(x, Wt, b2d)


def kernel(target, emb_table, W, b):
    target = target.astype(jnp.int32)
    x = _sc_gather_cols(target, emb_table.T)
    out_t = _tc_project_t(x, W.T, b.reshape(1, VOCAB))
    return out_t.T
